# Initial kernel scaffold; baseline (speedup 1.0000x reference)
import jax
import jax.numpy as jnp
from jax import lax
from jax.experimental import pallas as pl
from jax.experimental.pallas import tpu as pltpu
from jax.experimental.pallas import tpu_sc as plsc

M, J, I, L, W = 16, 2, 3, 2, 9
B = 16384
C = W * L
NC, NS, LN = 2, 16, 16
NW = NC * NS
RPW = B // NW
NG = RPW // LN

_OG, _OC, _OHW, _OHB, _OTW, _OTB = 0, 96, 192, 288, 384, 480


def _body(state_hbm, pk_hbm, out_hbm, sv, pv, dp, ov):
    wid = lax.axis_index("s") * NC + lax.axis_index("c")
    base = wid * RPW
    pltpu.sync_copy(state_hbm.at[pl.ds(base, RPW)], sv)
    pltpu.sync_copy(pk_hbm, pv)

    mi = jnp.arange(LN, dtype=jnp.int32)

    def pick(off):
        return plsc.load_gather(pv, [mi * 6 + off])

    g = [[1.0 / (1.0 + jnp.exp(-pick(_OG + j * L + l))) for l in range(L)]
         for j in range(J)]
    c = [[pick(_OC + j * L + l) for l in range(L)] for j in range(J)]
    hw = [[pick(_OHW + j * I + i) for i in range(I)] for j in range(J)]
    hb = [[pick(_OHB + j * I + i) for i in range(I)] for j in range(J)]
    tw = [[pick(_OTW + lp * I + i) for i in range(I)] for lp in range(L)]
    tb = [plsc.load_gather(pv, [mi * L + (_OTB + lp)]) for lp in range(L)]

    a = [(1.0 - g[0][l]) + (1.0 - g[1][l]) for l in range(L)]
    nb = [-2.0 * ((1.0 - g[0][l]) * c[0][l] + (1.0 - g[1][l]) * c[1][l])
          for l in range(L)]
    cc = sum((1.0 - g[j][l]) * c[j][l] * c[j][l]
             for j in range(J) for l in range(L))
    Rm = [[sum(tw[lp][i] * hw[j][i] for i in range(I)) for j in range(J)]
          for lp in range(L)]
    Sm = [[sum(tw[lp][i] * hb[j][i] for i in range(I)) for j in range(J)]
          for lp in range(L)]
    P = [[sum(Rm[lp][j] * g[j][l] for j in range(J)) for l in range(L)]
         for lp in range(L)]
    Gj = [g[j][0] + g[j][1] for j in range(J)]
    q = [sum(Sm[lp][j] * Gj[j] for j in range(J)) + tb[lp] for lp in range(L)]

    derived = [a[0], a[1], nb[0], nb[1], cc,
               P[0][0], P[0][1], P[1][0], P[1][1], q[0], q[1]]
    for k, v in enumerate(derived):
        dp[k, :] = v

    def group(gi, carry):
        rows = mi + gi * LN
        s = [[plsc.load_gather(sv, [rows, jnp.full((LN,), w * L + l, jnp.int32)])
              for l in range(L)] for w in range(W)]
        sq = [[s[w][l] * s[w][l] for l in range(L)] for w in range(W)]
        ow = [jnp.zeros((LN,), jnp.float32) for _ in range(W)]
        for m in range(M):
            a0, a1 = dp[0, m], dp[1, m]
            nb0, nb1 = dp[2, m], dp[3, m]
            ccm = dp[4, m]
            p00, p01, p10, p11 = dp[5, m], dp[6, m], dp[7, m], dp[8, m]
            q0, q1 = dp[9, m], dp[10, m]
            mn = (a0 * sq[0][0] + nb0 * s[0][0]) + \
                 (a1 * sq[0][1] + nb1 * s[0][1]) + ccm
            sb0, sb1 = s[0][0], s[0][1]
            for w in range(1, W):
                t = (a0 * sq[w][0] + nb0 * s[w][0]) + \
                    (a1 * sq[w][1] + nb1 * s[w][1]) + ccm
                lt = t < mn
                mn = jnp.where(lt, t, mn)
                sb0 = jnp.where(lt, s[w][0], sb0)
                sb1 = jnp.where(lt, s[w][1], sb1)
            cl0 = p00 * sb0 + p01 * sb1 + q0
            cl1 = p10 * sb0 + p11 * sb1 + q1
            conf = jnp.exp(-mn)
            for w in range(W):
                d0 = cl0 - s[w][0]
                d1 = cl1 - s[w][1]
                ow[w] = ow[w] + conf * jnp.exp(-(d0 * d0 + d1 * d1))
        for w in range(W):
            plsc.store_scatter(ov, [rows, jnp.full((LN,), w, jnp.int32)], ow[w])
        return carry

    lax.fori_loop(0, NG, group, 0)
    pltpu.sync_copy(ov, out_hbm.at[pl.ds(base, RPW)])


@jax.jit
def kernel(state, constants, gammas, head_w, head_b, tail_w, tail_b):
    pk = jnp.concatenate([
        gammas.reshape(-1), constants.reshape(-1),
        head_w.reshape(-1), head_b.reshape(-1),
        tail_w.reshape(-1), tail_b.reshape(-1),
    ]).astype(jnp.float32)
    mesh = plsc.VectorSubcoreMesh(
        core_axis_name="c", subcore_axis_name="s",
        num_cores=NC, num_subcores=NS)
    f = pl.kernel(
        _body,
        out_type=jax.ShapeDtypeStruct((B, W), jnp.float32),
        mesh=mesh,
        scratch_types=[
            pltpu.VMEM((RPW, C), jnp.float32),
            pltpu.VMEM((512,), jnp.float32),
            pltpu.VMEM((11, LN), jnp.float32),
            pltpu.VMEM((RPW, W), jnp.float32),
        ],
    )
    return f(state, pk)


# speedup vs baseline: 36.8908x; 36.8908x over previous
import jax
import jax.numpy as jnp
from jax import lax
from jax.experimental import pallas as pl
from jax.experimental.pallas import tpu as pltpu
from jax.experimental.pallas import tpu_sc as plsc

M, J, I, L, W = 16, 2, 3, 2, 9
B = 16384
C = W * L
NC, NS, LN = 2, 16, 16
NW = NC * NS
RPW = B // NW
NG = RPW // LN

_OG, _OC, _OHW, _OHB, _OTW, _OTB = 0, 96, 192, 288, 384, 480


def _body(state_hbm, pk_hbm, out_hbm, sv, pv, ov):
    wid = lax.axis_index("s") * NC + lax.axis_index("c")
    base = wid * RPW
    pltpu.sync_copy(state_hbm.at[pl.ds(base * C, RPW * C)], sv)
    pltpu.sync_copy(pk_hbm, pv)

    mi = jnp.arange(LN, dtype=jnp.int32)

    def pick(off):
        return plsc.load_gather(pv, [mi * 6 + off])

    g = [[1.0 / (1.0 + jnp.exp(-pick(_OG + j * L + l))) for l in range(L)]
         for j in range(J)]
    c = [[pick(_OC + j * L + l) for l in range(L)] for j in range(J)]
    hw = [[pick(_OHW + j * I + i) for i in range(I)] for j in range(J)]
    hb = [[pick(_OHB + j * I + i) for i in range(I)] for j in range(J)]
    tw = [[pick(_OTW + lp * I + i) for i in range(I)] for lp in range(L)]
    tb = [plsc.load_gather(pv, [mi * L + (_OTB + lp)]) for lp in range(L)]

    a = [(1.0 - g[0][l]) + (1.0 - g[1][l]) for l in range(L)]
    nb = [-2.0 * ((1.0 - g[0][l]) * c[0][l] + (1.0 - g[1][l]) * c[1][l])
          for l in range(L)]
    cc = sum((1.0 - g[j][l]) * c[j][l] * c[j][l]
             for j in range(J) for l in range(L))
    Rm = [[sum(tw[lp][i] * hw[j][i] for i in range(I)) for j in range(J)]
          for lp in range(L)]
    Sm = [[sum(tw[lp][i] * hb[j][i] for i in range(I)) for j in range(J)]
          for lp in range(L)]
    P = [[sum(Rm[lp][j] * g[j][l] for j in range(J)) for l in range(L)]
         for lp in range(L)]
    Gj = [g[j][0] + g[j][1] for j in range(J)]
    q = [sum(Sm[lp][j] * Gj[j] for j in range(J)) + tb[lp] for lp in range(L)]

    def group(gi, carry):
        rows = mi + gi * LN
        rb = rows * C
        ob = rows * W
        s = [[plsc.load_gather(sv, [rb + (w * L + l)])
              for l in range(L)] for w in range(W)]
        sq = [[s[w][l] * s[w][l] for l in range(L)] for w in range(W)]
        ow = [jnp.zeros((LN,), jnp.float32) for _ in range(W)]
        for m in range(M):
            a0, a1 = a[0][m], a[1][m]
            nb0, nb1 = nb[0][m], nb[1][m]
            ccm = cc[m]
            p00, p01 = P[0][0][m], P[0][1][m]
            p10, p11 = P[1][0][m], P[1][1][m]
            q0, q1 = q[0][m], q[1][m]
            mn = (a0 * sq[0][0] + nb0 * s[0][0]) + \
                 (a1 * sq[0][1] + nb1 * s[0][1]) + ccm
            sb0, sb1 = s[0][0], s[0][1]
            for w in range(1, W):
                t = (a0 * sq[w][0] + nb0 * s[w][0]) + \
                    (a1 * sq[w][1] + nb1 * s[w][1]) + ccm
                lt = t < mn
                mn = jnp.where(lt, t, mn)
                sb0 = jnp.where(lt, s[w][0], sb0)
                sb1 = jnp.where(lt, s[w][1], sb1)
            cl0 = p00 * sb0 + p01 * sb1 + q0
            cl1 = p10 * sb0 + p11 * sb1 + q1
            conf = jnp.exp(-mn)
            for w in range(W):
                d0 = cl0 - s[w][0]
                d1 = cl1 - s[w][1]
                ow[w] = ow[w] + conf * jnp.exp(-(d0 * d0 + d1 * d1))
        for w in range(W):
            plsc.store_scatter(ov, [ob + w], ow[w])
        return carry

    lax.fori_loop(0, NG, group, 0)
    pltpu.sync_copy(ov, out_hbm.at[pl.ds(base * W, RPW * W)])


@jax.jit
def kernel(state, constants, gammas, head_w, head_b, tail_w, tail_b):
    pk = jnp.concatenate([
        gammas.reshape(-1), constants.reshape(-1),
        head_w.reshape(-1), head_b.reshape(-1),
        tail_w.reshape(-1), tail_b.reshape(-1),
    ]).astype(jnp.float32)
    mesh = plsc.VectorSubcoreMesh(
        core_axis_name="c", subcore_axis_name="s",
        num_cores=NC, num_subcores=NS)
    f = pl.kernel(
        _body,
        out_type=jax.ShapeDtypeStruct((B * W,), jnp.float32),
        mesh=mesh,
        compiler_params=pltpu.CompilerParams(needs_layout_passes=False),
        scratch_types=[
            pltpu.VMEM((RPW * C,), jnp.float32),
            pltpu.VMEM((512,), jnp.float32),
            pltpu.VMEM((RPW * W,), jnp.float32),
        ],
    )
    return f(state.reshape(-1), pk).reshape(B, W)
